# baseline (speedup 1.0000x reference)
import functools

import jax
import jax.numpy as jnp
from jax import lax
from jax.experimental import pallas as pl
from jax.experimental.pallas import tpu as pltpu
from jax.experimental.pallas import tpu_sc as plsc

N = 10000
D = 128
E = 320000
NC, NS = 2, 16
NW = NC * NS
CPW = 80
CPH = CPW // 2
CHP = NW * CPW
N_PAD = N + 128
SL = 640
SL_LAST = N - SL * (NS - 1)

_mesh = lambda: plsc.VectorSubcoreMesh(core_axis_name="c", subcore_axis_name="s")


def _for_my_slice(s, fn):
    @pl.when(s < NS - 1)
    def _():
        def b(k, carry):
            fn(pl.multiple_of(s * SL + k * 128, 128), 128)
            return carry

        lax.fori_loop(0, SL // 128, b, 0)

    @pl.when(s == NS - 1)
    def _():
        base = (NS - 1) * SL
        for k in range(SL_LAST // 128):
            fn(base + k * 128, 128)
        fn(base + (SL_LAST // 128) * 128, SL_LAST % 128)


def _degree_partials(dst2):
    @functools.partial(
        pl.kernel,
        out_type=jax.ShapeDtypeStruct((NC, 1, N), jnp.float32),
        mesh=_mesh(),
        scratch_types=[
            pltpu.VMEM_SHARED((N_PAD,), jnp.float32),
            pltpu.VMEM((CPW, 128), jnp.int32),
            pltpu.VMEM((128,), jnp.float32),
            pltpu.VMEM((128,), jnp.float32),
            pltpu.SemaphoreType.DMA,
        ],
    )
    def k(dst_hbm, out_hbm, acc, didx_all, ones_v, stage, sem):
        c = lax.axis_index("c")
        s = lax.axis_index("s")
        w = s * NC + c
        for i in range(8):
            ones_v[pl.ds(i * 16, 16)] = jnp.ones((16,), jnp.float32)
            stage[pl.ds(i * 16, 16)] = jnp.zeros((16,), jnp.float32)

        _for_my_slice(s, lambda off, ln: pltpu.sync_copy(
            stage.at[pl.ds(0, ln)], acc.at[pl.ds(off, ln)]))
        plsc.subcore_barrier()

        base = w * CPW
        pltpu.sync_copy(dst_hbm.at[pl.ds(base, CPW)], didx_all)

        def body(j, carry):
            pltpu.async_copy(ones_v, acc.at[didx_all.at[j]], sem, add=True)
            return carry

        lax.fori_loop(0, CPW, body, 0)

        def drain(j, carry):
            pltpu.make_async_copy(ones_v, acc.at[didx_all.at[0]], sem).wait()
            return carry

        lax.fori_loop(0, CPW, drain, 0)
        plsc.subcore_barrier()

        def wb(off, ln):
            pltpu.sync_copy(acc.at[pl.ds(off, ln)], stage.at[pl.ds(0, ln)])
            pltpu.sync_copy(stage.at[pl.ds(0, ln)],
                            out_hbm.at[c].at[0].at[pl.ds(off, ln)])

        _for_my_slice(s, wb)

    return k(dst2)


def _propagate(xws, src2, dst2):
    @functools.partial(
        pl.kernel,
        out_type=jax.ShapeDtypeStruct((NC, N, D), jnp.float32),
        mesh=_mesh(),
        scratch_types=[
            pltpu.VMEM_SHARED((N_PAD, D), jnp.float32),
            pltpu.VMEM((CPH, 128), jnp.int32),
            pltpu.VMEM((CPW, 128), jnp.int32),
            pltpu.VMEM((2, 128, D), jnp.float32),
            pltpu.SemaphoreType.DMA,
            pltpu.SemaphoreType.DMA,
            pltpu.SemaphoreType.DMA,
            pltpu.SemaphoreType.DMA,
        ],
    )
    def k(xws_hbm, src_hbm, dst_hbm, out_hbm, acc, sidx_h, didx_all, rows,
          gsem0, gsem1, ssem0, ssem1):
        c = lax.axis_index("c")
        s = lax.axis_index("s")
        w = s * NC + c

        def zrow(i, carry):
            def zcol(j, c2):
                rows[0, i, pl.ds(j * 16, 16)] = jnp.zeros((16,), jnp.float32)
                return c2

            return lax.fori_loop(0, D // 16, zcol, carry)

        lax.fori_loop(0, 128, zrow, 0)

        _for_my_slice(s, lambda off, ln: pltpu.sync_copy(
            rows.at[0].at[pl.ds(0, ln)], acc.at[pl.ds(off, ln)]))
        plsc.subcore_barrier()

        base = w * CPW
        pltpu.sync_copy(dst_hbm.at[pl.ds(base, CPW)], didx_all)

        def gath(jloc, b, sem):
            pltpu.async_copy(xws_hbm.at[sidx_h.at[jloc]], rows.at[b], sem)

        def gath_wait(b, sem):
            pltpu.make_async_copy(xws_hbm.at[sidx_h.at[0]], rows.at[b],
                                  sem).wait()

        def scat(j, b, sem):
            pltpu.async_copy(rows.at[b], acc.at[didx_all.at[j]], sem, add=True)

        def scat_wait(b, sem):
            pltpu.make_async_copy(rows.at[b], acc.at[didx_all.at[0]],
                                  sem).wait()

        for h in range(2):
            hb = h * CPH
            pltpu.sync_copy(src_hbm.at[pl.ds(base + hb, CPH)], sidx_h)

            def inner(t, carry):
                gath(t, 0, gsem0)
                gath_wait(0, gsem0)
                pltpu.sync_copy(rows.at[0], acc.at[didx_all.at[hb + t]],
                                add=True)
                return carry

            lax.fori_loop(0, CPH, inner, 0)

        plsc.subcore_barrier()

        def wb(off, ln):
            pltpu.sync_copy(acc.at[pl.ds(off, ln)], rows.at[0].at[pl.ds(0, ln)])
            pltpu.sync_copy(rows.at[0].at[pl.ds(0, ln)],
                            out_hbm.at[c].at[pl.ds(off, ln)])

        _for_my_slice(s, wb)

    return k(xws, src2, dst2)


R = 1000


def _mm_scale_body(x_ref, w_ref, degp_ref, o_ref):
    d = degp_ref[:, 0] + degp_ref[:, 1] + 1.0
    dinv = lax.rsqrt(d)
    xw = jnp.dot(x_ref[...], w_ref[...], preferred_element_type=jnp.float32,
                 precision=lax.Precision.HIGHEST)
    o_ref[...] = dinv[:, None] * xw


def _mm_scale(x, w, degp):
    return pl.pallas_call(
        _mm_scale_body,
        grid=(N // R,),
        in_specs=[
            pl.BlockSpec((R, D), lambda i: (i, 0)),
            pl.BlockSpec((D, D), lambda i: (0, 0)),
            pl.BlockSpec((R, NC), lambda i: (i, 0)),
        ],
        out_specs=pl.BlockSpec((R, D), lambda i: (i, 0)),
        out_shape=jax.ShapeDtypeStruct((N, D), jnp.float32),
    )(x, w, degp)


def _mid_body(p_ref, xws_ref, degp_ref, b_ref, w2_ref, o_ref):
    d = degp_ref[:, 0] + degp_ref[:, 1] + 1.0
    dinv = lax.rsqrt(d)
    ssum = p_ref[0] + p_ref[1] + xws_ref[...]
    h = jnp.maximum(dinv[:, None] * ssum + b_ref[0, :][None, :], 0.0)
    hw = jnp.dot(h, w2_ref[...], preferred_element_type=jnp.float32,
                 precision=lax.Precision.HIGHEST)
    o_ref[...] = dinv[:, None] * hw


def _mid(p, xws, degp, b1, w2):
    return pl.pallas_call(
        _mid_body,
        grid=(N // R,),
        in_specs=[
            pl.BlockSpec((NC, R, D), lambda i: (0, i, 0)),
            pl.BlockSpec((R, D), lambda i: (i, 0)),
            pl.BlockSpec((R, NC), lambda i: (i, 0)),
            pl.BlockSpec((1, D), lambda i: (0, 0)),
            pl.BlockSpec((D, D), lambda i: (0, 0)),
        ],
        out_specs=pl.BlockSpec((R, D), lambda i: (i, 0)),
        out_shape=jax.ShapeDtypeStruct((N, D), jnp.float32),
    )(p, xws, degp, b1, w2)


def _final_body(q_ref, xws2_ref, degp_ref, b_ref, o_ref):
    d = degp_ref[:, 0] + degp_ref[:, 1] + 1.0
    dinv = lax.rsqrt(d)
    ssum = q_ref[0] + q_ref[1] + xws2_ref[...]
    o_ref[...] = jnp.maximum(dinv[:, None] * ssum + b_ref[0, :][None, :], 0.0)


def _final(q, xws2, degp, b2):
    return pl.pallas_call(
        _final_body,
        grid=(N // R,),
        in_specs=[
            pl.BlockSpec((NC, R, D), lambda i: (0, i, 0)),
            pl.BlockSpec((R, D), lambda i: (i, 0)),
            pl.BlockSpec((R, NC), lambda i: (i, 0)),
            pl.BlockSpec((1, D), lambda i: (0, 0)),
        ],
        out_specs=pl.BlockSpec((R, D), lambda i: (i, 0)),
        out_shape=jax.ShapeDtypeStruct((N, D), jnp.float32),
    )(q, xws2, degp, b2)


def kernel(x, edge_index, W1, b1, W2, b2):
    pad = CHP * 128 - E
    pad_dst = N + (jnp.arange(pad, dtype=jnp.int32) % 128)
    src2 = jnp.concatenate(
        [edge_index[0], jnp.zeros((pad,), jnp.int32)]).reshape(CHP, 128)
    dst2 = jnp.concatenate([edge_index[1], pad_dst]).reshape(CHP, 128)
    b1r = b1.reshape(1, D)
    b2r = b2.reshape(1, D)

    degp = _degree_partials(dst2).reshape(NC, N).T
    xws1 = _mm_scale(x, W1, degp)
    p = _propagate(xws1, src2, dst2)
    xws2 = _mid(p, xws1, degp, b1r, W2)
    q = _propagate(xws2, src2, dst2)
    return _final(q, xws2, degp, b2r)


# speedup vs baseline: 1.0270x; 1.0270x over previous
import functools

import jax
import jax.numpy as jnp
from jax import lax
from jax.experimental import pallas as pl
from jax.experimental.pallas import tpu as pltpu
from jax.experimental.pallas import tpu_sc as plsc

N = 10000
D = 128
E = 320000
NC, NS = 2, 16
NW = NC * NS
CPW = 80
CPH = CPW // 2
CHP = NW * CPW
N_PAD = N + 128
SL = 640
SL_LAST = N - SL * (NS - 1)

_mesh = lambda: plsc.VectorSubcoreMesh(core_axis_name="c", subcore_axis_name="s")


def _for_my_slice(s, fn):
    @pl.when(s < NS - 1)
    def _():
        def b(k, carry):
            fn(pl.multiple_of(s * SL + k * 128, 128), 128)
            return carry

        lax.fori_loop(0, SL // 128, b, 0)

    @pl.when(s == NS - 1)
    def _():
        base = (NS - 1) * SL
        for k in range(SL_LAST // 128):
            fn(base + k * 128, 128)
        fn(base + (SL_LAST // 128) * 128, SL_LAST % 128)


def _degree_partials(dst2):
    @functools.partial(
        pl.kernel,
        out_type=jax.ShapeDtypeStruct((NC, 1, N), jnp.float32),
        mesh=_mesh(),
        scratch_types=[
            pltpu.VMEM_SHARED((N_PAD,), jnp.float32),
            pltpu.VMEM((CPW, 128), jnp.int32),
            pltpu.VMEM((128,), jnp.float32),
            pltpu.VMEM((128,), jnp.float32),
            pltpu.SemaphoreType.DMA,
        ],
    )
    def k(dst_hbm, out_hbm, acc, didx_all, ones_v, stage, sem):
        c = lax.axis_index("c")
        s = lax.axis_index("s")
        w = s * NC + c
        for i in range(8):
            ones_v[pl.ds(i * 16, 16)] = jnp.ones((16,), jnp.float32)
            stage[pl.ds(i * 16, 16)] = jnp.zeros((16,), jnp.float32)

        _for_my_slice(s, lambda off, ln: pltpu.sync_copy(
            stage.at[pl.ds(0, ln)], acc.at[pl.ds(off, ln)]))
        plsc.subcore_barrier()

        base = w * CPW
        pltpu.sync_copy(dst_hbm.at[pl.ds(base, CPW)], didx_all)

        def body(j, carry):
            pltpu.async_copy(ones_v, acc.at[didx_all.at[j]], sem, add=True)
            return carry

        lax.fori_loop(0, CPW, body, 0)

        def drain(j, carry):
            pltpu.make_async_copy(ones_v, acc.at[didx_all.at[0]], sem).wait()
            return carry

        lax.fori_loop(0, CPW, drain, 0)
        plsc.subcore_barrier()

        def wb(off, ln):
            pltpu.sync_copy(acc.at[pl.ds(off, ln)], stage.at[pl.ds(0, ln)])
            pltpu.sync_copy(stage.at[pl.ds(0, ln)],
                            out_hbm.at[c].at[0].at[pl.ds(off, ln)])

        _for_my_slice(s, wb)

    return k(dst2)


def _propagate(xws, src2, dst2):
    @functools.partial(
        pl.kernel,
        out_type=jax.ShapeDtypeStruct((NC, N, D), jnp.float32),
        mesh=_mesh(),
        scratch_types=[
            pltpu.VMEM_SHARED((N_PAD, D), jnp.float32),
            pltpu.VMEM((128, D), jnp.float32),
            pltpu.VMEM((128, D), jnp.float32),
            pltpu.VMEM((128,), jnp.int32),
            pltpu.VMEM((128,), jnp.int32),
            pltpu.VMEM((128,), jnp.int32),
            pltpu.VMEM((128,), jnp.int32),
            pltpu.SemaphoreType.DMA,
            pltpu.SemaphoreType.DMA,
            pltpu.SemaphoreType.DMA,
            pltpu.SemaphoreType.DMA,
        ],
    )
    def k(xws_hbm, src_hbm, dst_hbm, out_hbm, acc, rows_a, rows_b,
          sidx_a, sidx_b, didx_a, didx_b, gsem_a, gsem_b, ssem_a, ssem_b):
        c = lax.axis_index("c")
        s = lax.axis_index("s")
        w = s * NC + c

        def zrow(i, carry):
            def zcol(j, c2):
                rows_a[i, pl.ds(j * 16, 16)] = jnp.zeros((16,), jnp.float32)
                return c2

            return lax.fori_loop(0, D // 16, zcol, carry)

        lax.fori_loop(0, 128, zrow, 0)

        _for_my_slice(s, lambda off, ln: pltpu.sync_copy(
            rows_a.at[pl.ds(0, ln)], acc.at[pl.ds(off, ln)]))
        plsc.subcore_barrier()

        base = w * CPW

        def gath(rows, sidx, sem):
            pltpu.async_copy(xws_hbm.at[sidx], rows, sem)

        def gath_wait(rows, sidx, sem):
            pltpu.make_async_copy(xws_hbm.at[sidx], rows, sem).wait()

        def scat(rows, didx, sem):
            pltpu.async_copy(rows, acc.at[didx], sem, add=True)

        def scat_wait(rows, didx, sem):
            pltpu.make_async_copy(rows, acc.at[didx], sem).wait()

        def inner(t, carry):
            j0 = base + 2 * t
            j1 = j0 + 1
            pltpu.sync_copy(src_hbm.at[j0], sidx_a)
            pltpu.sync_copy(dst_hbm.at[j0], didx_a)

            @pl.when(t > 0)
            def _():
                scat_wait(rows_b, didx_b, ssem_b)

            gath(rows_a, sidx_a, gsem_a)
            pltpu.sync_copy(src_hbm.at[j1], sidx_b)
            pltpu.sync_copy(dst_hbm.at[j1], didx_b)
            gath_wait(rows_a, sidx_a, gsem_a)
            scat(rows_a, didx_a, ssem_a)
            gath(rows_b, sidx_b, gsem_b)
            gath_wait(rows_b, sidx_b, gsem_b)
            scat_wait(rows_a, didx_a, ssem_a)
            scat(rows_b, didx_b, ssem_b)
            return carry

        lax.fori_loop(0, CPW // 2, inner, 0)
        scat_wait(rows_b, didx_b, ssem_b)
        plsc.subcore_barrier()

        def wb(off, ln):
            pltpu.sync_copy(acc.at[pl.ds(off, ln)], rows_a.at[pl.ds(0, ln)])
            pltpu.sync_copy(rows_a.at[pl.ds(0, ln)],
                            out_hbm.at[c].at[pl.ds(off, ln)])

        _for_my_slice(s, wb)

    return k(xws, src2, dst2)


R = 1000


def _mm_scale_body(x_ref, w_ref, degp_ref, o_ref):
    d = degp_ref[:, 0] + degp_ref[:, 1] + 1.0
    dinv = lax.rsqrt(d)
    xw = jnp.dot(x_ref[...], w_ref[...], preferred_element_type=jnp.float32,
                 precision=lax.Precision.HIGHEST)
    o_ref[...] = dinv[:, None] * xw


def _mm_scale(x, w, degp):
    return pl.pallas_call(
        _mm_scale_body,
        grid=(N // R,),
        in_specs=[
            pl.BlockSpec((R, D), lambda i: (i, 0)),
            pl.BlockSpec((D, D), lambda i: (0, 0)),
            pl.BlockSpec((R, NC), lambda i: (i, 0)),
        ],
        out_specs=pl.BlockSpec((R, D), lambda i: (i, 0)),
        out_shape=jax.ShapeDtypeStruct((N, D), jnp.float32),
    )(x, w, degp)


def _mid_body(p_ref, xws_ref, degp_ref, b_ref, w2_ref, o_ref):
    d = degp_ref[:, 0] + degp_ref[:, 1] + 1.0
    dinv = lax.rsqrt(d)
    ssum = p_ref[0] + p_ref[1] + xws_ref[...]
    h = jnp.maximum(dinv[:, None] * ssum + b_ref[0, :][None, :], 0.0)
    hw = jnp.dot(h, w2_ref[...], preferred_element_type=jnp.float32,
                 precision=lax.Precision.HIGHEST)
    o_ref[...] = dinv[:, None] * hw


def _mid(p, xws, degp, b1, w2):
    return pl.pallas_call(
        _mid_body,
        grid=(N // R,),
        in_specs=[
            pl.BlockSpec((NC, R, D), lambda i: (0, i, 0)),
            pl.BlockSpec((R, D), lambda i: (i, 0)),
            pl.BlockSpec((R, NC), lambda i: (i, 0)),
            pl.BlockSpec((1, D), lambda i: (0, 0)),
            pl.BlockSpec((D, D), lambda i: (0, 0)),
        ],
        out_specs=pl.BlockSpec((R, D), lambda i: (i, 0)),
        out_shape=jax.ShapeDtypeStruct((N, D), jnp.float32),
    )(p, xws, degp, b1, w2)


def _final_body(q_ref, xws2_ref, degp_ref, b_ref, o_ref):
    d = degp_ref[:, 0] + degp_ref[:, 1] + 1.0
    dinv = lax.rsqrt(d)
    ssum = q_ref[0] + q_ref[1] + xws2_ref[...]
    o_ref[...] = jnp.maximum(dinv[:, None] * ssum + b_ref[0, :][None, :], 0.0)


def _final(q, xws2, degp, b2):
    return pl.pallas_call(
        _final_body,
        grid=(N // R,),
        in_specs=[
            pl.BlockSpec((NC, R, D), lambda i: (0, i, 0)),
            pl.BlockSpec((R, D), lambda i: (i, 0)),
            pl.BlockSpec((R, NC), lambda i: (i, 0)),
            pl.BlockSpec((1, D), lambda i: (0, 0)),
        ],
        out_specs=pl.BlockSpec((R, D), lambda i: (i, 0)),
        out_shape=jax.ShapeDtypeStruct((N, D), jnp.float32),
    )(q, xws2, degp, b2)


def kernel(x, edge_index, W1, b1, W2, b2):
    pad = CHP * 128 - E
    pad_dst = N + (jnp.arange(pad, dtype=jnp.int32) % 128)
    src2 = jnp.concatenate(
        [edge_index[0], jnp.zeros((pad,), jnp.int32)]).reshape(CHP, 128)
    dst2 = jnp.concatenate([edge_index[1], pad_dst]).reshape(CHP, 128)
    b1r = b1.reshape(1, D)
    b2r = b2.reshape(1, D)

    degp = _degree_partials(dst2).reshape(NC, N).T
    xws1 = _mm_scale(x, W1, degp)
    p = _propagate(xws1, src2, dst2)
    xws2 = _mid(p, xws1, degp, b1r, W2)
    q = _propagate(xws2, src2, dst2)
    return _final(q, xws2, degp, b2r)


# speedup vs baseline: 1.9129x; 1.8626x over previous
import functools

import jax
import jax.numpy as jnp
from jax import lax
from jax.experimental import pallas as pl
from jax.experimental.pallas import tpu as pltpu
from jax.experimental.pallas import tpu_sc as plsc

N = 10000
D = 128
E = 320000
CH = E // 128
NC, NS = 2, 16
NW = NC * NS
ROWS_PER_W = CH // NW
EXTRA = CH % NW
SL = 640
SL_LAST = N - SL * (NS - 1)

_mesh = lambda: plsc.VectorSubcoreMesh(core_axis_name="c", subcore_axis_name="s")


def _for_my_slice(s, fn):
    @pl.when(s < NS - 1)
    def _():
        def b(k, carry):
            fn(pl.multiple_of(s * SL + k * 128, 128), 128)
            return carry

        lax.fori_loop(0, SL // 128, b, 0)

    @pl.when(s == NS - 1)
    def _():
        base = (NS - 1) * SL
        for k in range(SL_LAST // 128):
            fn(base + k * 128, 128)
        fn(base + (SL_LAST // 128) * 128, SL_LAST % 128)


def _degree_partials(dst2):
    @functools.partial(
        pl.kernel,
        out_type=jax.ShapeDtypeStruct((NC, 1, N), jnp.float32),
        mesh=_mesh(),
        scratch_types=[
            pltpu.VMEM_SHARED((N,), jnp.float32),
            pltpu.VMEM((128,), jnp.int32),
            pltpu.VMEM((128,), jnp.float32),
            pltpu.VMEM((128,), jnp.float32),
        ],
    )
    def k(dst_hbm, out_hbm, acc, didx, ones_v, stage):
        c = lax.axis_index("c")
        s = lax.axis_index("s")
        w = s * NC + c
        for i in range(8):
            ones_v[pl.ds(i * 16, 16)] = jnp.ones((16,), jnp.float32)
            stage[pl.ds(i * 16, 16)] = jnp.zeros((16,), jnp.float32)

        _for_my_slice(s, lambda off, ln: pltpu.sync_copy(
            stage.at[pl.ds(0, ln)], acc.at[pl.ds(off, ln)]))
        plsc.subcore_barrier()

        base = w * ROWS_PER_W + jnp.minimum(w, EXTRA)
        n = ROWS_PER_W + jnp.where(w < EXTRA, 1, 0)

        def body(j, carry):
            pltpu.sync_copy(dst_hbm.at[base + j], didx)
            pltpu.sync_copy(ones_v, acc.at[didx], add=True)
            return carry

        lax.fori_loop(0, n, body, 0)
        plsc.subcore_barrier()

        def wb(off, ln):
            pltpu.sync_copy(acc.at[pl.ds(off, ln)], stage.at[pl.ds(0, ln)])
            pltpu.sync_copy(stage.at[pl.ds(0, ln)],
                            out_hbm.at[c].at[0].at[pl.ds(off, ln)])

        _for_my_slice(s, wb)

    return k(dst2)


def _propagate(xws, src2, dst2):
    @functools.partial(
        pl.kernel,
        out_type=jax.ShapeDtypeStruct((NC, N, D), jnp.float32),
        mesh=_mesh(),
        scratch_types=[
            pltpu.VMEM_SHARED((N, D), jnp.float32),
            pltpu.VMEM((128,), jnp.int32),
            pltpu.VMEM((128,), jnp.int32),
            pltpu.VMEM((128, D), jnp.float32),
            pltpu.SemaphoreType.DMA,
        ],
    )
    def k(xws_hbm, src_hbm, dst_hbm, out_hbm, acc, sidx, didx, rows, sem):
        c = lax.axis_index("c")
        s = lax.axis_index("s")
        w = s * NC + c

        def zrow(i, carry):
            def zcol(j, c2):
                rows[i, pl.ds(j * 16, 16)] = jnp.zeros((16,), jnp.float32)
                return c2

            return lax.fori_loop(0, 8, zcol, carry)

        lax.fori_loop(0, 128, zrow, 0)

        _for_my_slice(s, lambda off, ln: pltpu.sync_copy(
            rows.at[pl.ds(0, ln)], acc.at[pl.ds(off, ln)]))
        plsc.subcore_barrier()

        base = w * ROWS_PER_W + jnp.minimum(w, EXTRA)
        n = ROWS_PER_W + jnp.where(w < EXTRA, 1, 0)

        def body(j, carry):
            pltpu.sync_copy(src_hbm.at[base + j], sidx)
            pltpu.sync_copy(dst_hbm.at[base + j], didx)
            pltpu.async_copy(xws_hbm.at[sidx], rows, sem).wait()
            pltpu.sync_copy(rows, acc.at[didx], add=True)
            return carry

        lax.fori_loop(0, n, body, 0)
        plsc.subcore_barrier()

        def wb(off, ln):
            pltpu.sync_copy(acc.at[pl.ds(off, ln)], rows.at[pl.ds(0, ln)])
            pltpu.sync_copy(rows.at[pl.ds(0, ln)],
                            out_hbm.at[c].at[pl.ds(off, ln)])

        _for_my_slice(s, wb)

    return k(xws, src2, dst2)


R = 1000


def _mm_scale_body(x_ref, w_ref, degp_ref, o_ref):
    d = degp_ref[:, 0] + degp_ref[:, 1] + 1.0
    dinv = lax.rsqrt(d)
    xw = jnp.dot(x_ref[...], w_ref[...], preferred_element_type=jnp.float32,
                 precision=lax.Precision.HIGHEST)
    o_ref[...] = dinv[:, None] * xw


def _mm_scale(x, w, degp):
    return pl.pallas_call(
        _mm_scale_body,
        grid=(N // R,),
        in_specs=[
            pl.BlockSpec((R, D), lambda i: (i, 0)),
            pl.BlockSpec((D, D), lambda i: (0, 0)),
            pl.BlockSpec((R, NC), lambda i: (i, 0)),
        ],
        out_specs=pl.BlockSpec((R, D), lambda i: (i, 0)),
        out_shape=jax.ShapeDtypeStruct((N, D), jnp.float32),
    )(x, w, degp)


def _mid_body(p_ref, xws_ref, degp_ref, b_ref, w2_ref, o_ref):
    d = degp_ref[:, 0] + degp_ref[:, 1] + 1.0
    dinv = lax.rsqrt(d)
    ssum = p_ref[0] + p_ref[1] + xws_ref[...]
    h = jnp.maximum(dinv[:, None] * ssum + b_ref[0, :][None, :], 0.0)
    hw = jnp.dot(h, w2_ref[...], preferred_element_type=jnp.float32,
                 precision=lax.Precision.HIGHEST)
    o_ref[...] = dinv[:, None] * hw


def _mid(p, xws, degp, b1, w2):
    return pl.pallas_call(
        _mid_body,
        grid=(N // R,),
        in_specs=[
            pl.BlockSpec((NC, R, D), lambda i: (0, i, 0)),
            pl.BlockSpec((R, D), lambda i: (i, 0)),
            pl.BlockSpec((R, NC), lambda i: (i, 0)),
            pl.BlockSpec((1, D), lambda i: (0, 0)),
            pl.BlockSpec((D, D), lambda i: (0, 0)),
        ],
        out_specs=pl.BlockSpec((R, D), lambda i: (i, 0)),
        out_shape=jax.ShapeDtypeStruct((N, D), jnp.float32),
    )(p, xws, degp, b1, w2)


def _final_body(q_ref, xws2_ref, degp_ref, b_ref, o_ref):
    d = degp_ref[:, 0] + degp_ref[:, 1] + 1.0
    dinv = lax.rsqrt(d)
    ssum = q_ref[0] + q_ref[1] + xws2_ref[...]
    o_ref[...] = jnp.maximum(dinv[:, None] * ssum + b_ref[0, :][None, :], 0.0)


def _final(q, xws2, degp, b2):
    return pl.pallas_call(
        _final_body,
        grid=(N // R,),
        in_specs=[
            pl.BlockSpec((NC, R, D), lambda i: (0, i, 0)),
            pl.BlockSpec((R, D), lambda i: (i, 0)),
            pl.BlockSpec((R, NC), lambda i: (i, 0)),
            pl.BlockSpec((1, D), lambda i: (0, 0)),
        ],
        out_specs=pl.BlockSpec((R, D), lambda i: (i, 0)),
        out_shape=jax.ShapeDtypeStruct((N, D), jnp.float32),
    )(q, xws2, degp, b2)


def kernel(x, edge_index, W1, b1, W2, b2):
    src2 = edge_index[0].reshape(CH, 128)
    dst2 = edge_index[1].reshape(CH, 128)
    b1r = b1.reshape(1, D)
    b2r = b2.reshape(1, D)

    degp = _degree_partials(dst2).reshape(NC, N).T
    xws1 = _mm_scale(x, W1, degp)
    p = _propagate(xws1, src2, dst2)
    xws2 = _mid(p, xws1, degp, b1r, W2)
    q = _propagate(xws2, src2, dst2)
    return _final(q, xws2, degp, b2r)


# speedup vs baseline: 2.8642x; 1.4973x over previous
import functools

import jax
import jax.numpy as jnp
from jax import lax
from jax.experimental import pallas as pl
from jax.experimental.pallas import tpu as pltpu
from jax.experimental.pallas import tpu_sc as plsc

N = 10000
D = 128
E = 320000
NC, NS = 2, 16
NW = NC * NS
CPW = 80
CPH = CPW // 2
CHP = NW * CPW
N_PAD = N + 128
SL = 640
SL_LAST = N - SL * (NS - 1)

_mesh = lambda: plsc.VectorSubcoreMesh(core_axis_name="c", subcore_axis_name="s")


def _for_my_slice(s, fn):
    @pl.when(s < NS - 1)
    def _():
        def b(k, carry):
            fn(pl.multiple_of(s * SL + k * 128, 128), 128)
            return carry

        lax.fori_loop(0, SL // 128, b, 0)

    @pl.when(s == NS - 1)
    def _():
        base = (NS - 1) * SL
        for k in range(SL_LAST // 128):
            fn(base + k * 128, 128)
        fn(base + (SL_LAST // 128) * 128, SL_LAST % 128)


def _degree_partials(dst2):
    @functools.partial(
        pl.kernel,
        out_type=jax.ShapeDtypeStruct((NC, 1, N), jnp.float32),
        mesh=_mesh(),
        scratch_types=[
            pltpu.VMEM_SHARED((N_PAD,), jnp.float32),
            pltpu.VMEM((CPW, 128), jnp.int32),
            pltpu.VMEM((128,), jnp.float32),
            pltpu.VMEM((128,), jnp.float32),
            pltpu.SemaphoreType.DMA,
        ],
    )
    def k(dst_hbm, out_hbm, acc, didx_all, ones_v, stage, sem):
        c = lax.axis_index("c")
        s = lax.axis_index("s")
        w = s * NC + c
        for i in range(8):
            ones_v[pl.ds(i * 16, 16)] = jnp.ones((16,), jnp.float32)
            stage[pl.ds(i * 16, 16)] = jnp.zeros((16,), jnp.float32)

        _for_my_slice(s, lambda off, ln: pltpu.sync_copy(
            stage.at[pl.ds(0, ln)], acc.at[pl.ds(off, ln)]))
        plsc.subcore_barrier()

        base = w * CPW
        pltpu.sync_copy(dst_hbm.at[pl.ds(base, CPW)], didx_all)

        def body(j, carry):
            pltpu.async_copy(ones_v, acc.at[didx_all.at[j]], sem, add=True)
            return carry

        lax.fori_loop(0, CPW, body, 0)

        def drain(j, carry):
            pltpu.make_async_copy(ones_v, acc.at[didx_all.at[0]], sem).wait()
            return carry

        lax.fori_loop(0, CPW, drain, 0)
        plsc.subcore_barrier()

        def wb(off, ln):
            pltpu.sync_copy(acc.at[pl.ds(off, ln)], stage.at[pl.ds(0, ln)])
            pltpu.sync_copy(stage.at[pl.ds(0, ln)],
                            out_hbm.at[c].at[0].at[pl.ds(off, ln)])

        _for_my_slice(s, wb)

    return k(dst2)


def _propagate(xws, src2, dst2):
    @functools.partial(
        pl.kernel,
        out_type=jax.ShapeDtypeStruct((NC, N, D), jnp.float32),
        mesh=_mesh(),
        scratch_types=[
            pltpu.VMEM_SHARED((N_PAD, D), jnp.float32),
            pltpu.VMEM((128, D), jnp.float32),
            pltpu.VMEM((128, D), jnp.float32),
            pltpu.VMEM((128,), jnp.int32),
            pltpu.VMEM((128,), jnp.int32),
            pltpu.VMEM((128,), jnp.int32),
            pltpu.VMEM((128,), jnp.int32),
            pltpu.SemaphoreType.DMA,
            pltpu.SemaphoreType.DMA,
            pltpu.SemaphoreType.DMA,
            pltpu.SemaphoreType.DMA,
        ],
    )
    def k(xws_hbm, src_hbm, dst_hbm, out_hbm, acc, rows_a, rows_b,
          sidx_a, sidx_b, didx_a, didx_b, gsem_a, gsem_b, ssem_a, ssem_b):
        c = lax.axis_index("c")
        s = lax.axis_index("s")
        w = s * NC + c

        def zrow(i, carry):
            def zcol(j, c2):
                rows_a[i, pl.ds(j * 16, 16)] = jnp.zeros((16,), jnp.float32)
                return c2

            return lax.fori_loop(0, D // 16, zcol, carry)

        lax.fori_loop(0, 128, zrow, 0)

        _for_my_slice(s, lambda off, ln: pltpu.sync_copy(
            rows_a.at[pl.ds(0, ln)], acc.at[pl.ds(off, ln)]))
        plsc.subcore_barrier()

        base = w * CPW

        def gath(rows, sidx, sem):
            pltpu.async_copy(xws_hbm.at[sidx], rows, sem)

        def gath_wait(rows, sidx, sem):
            pltpu.make_async_copy(xws_hbm.at[sidx], rows, sem).wait()

        def scat(rows, didx, sem):
            pltpu.async_copy(rows, acc.at[didx], sem, add=True)

        def scat_wait(rows, didx, sem):
            pltpu.make_async_copy(rows, acc.at[didx], sem).wait()

        def inner(t, carry):
            j0 = base + 2 * t
            j1 = j0 + 1
            pltpu.sync_copy(src_hbm.at[j0], sidx_a)
            pltpu.sync_copy(dst_hbm.at[j0], didx_a)

            @pl.when(t > 0)
            def _():
                scat_wait(rows_b, didx_b, ssem_b)

            gath(rows_a, sidx_a, gsem_a)
            pltpu.sync_copy(src_hbm.at[j1], sidx_b)
            pltpu.sync_copy(dst_hbm.at[j1], didx_b)
            gath_wait(rows_a, sidx_a, gsem_a)
            scat(rows_a, didx_a, ssem_a)
            gath(rows_b, sidx_b, gsem_b)
            gath_wait(rows_b, sidx_b, gsem_b)
            scat_wait(rows_a, didx_a, ssem_a)
            scat(rows_b, didx_b, ssem_b)
            return carry

        lax.fori_loop(0, CPW // 2, inner, 0)
        scat_wait(rows_b, didx_b, ssem_b)
        plsc.subcore_barrier()

        def wb(off, ln):
            pltpu.sync_copy(acc.at[pl.ds(off, ln)], rows_a.at[pl.ds(0, ln)])
            pltpu.sync_copy(rows_a.at[pl.ds(0, ln)],
                            out_hbm.at[c].at[pl.ds(off, ln)])

        _for_my_slice(s, wb)

    return k(xws, src2, dst2)


R = 1000


def _mm_scale_body(x_ref, w_ref, degp_ref, o_ref):
    d = degp_ref[:, 0] + degp_ref[:, 1] + 1.0
    dinv = lax.rsqrt(d)
    xw = jnp.dot(x_ref[...], w_ref[...], preferred_element_type=jnp.float32,
                 precision=lax.Precision.HIGHEST)
    o_ref[...] = dinv[:, None] * xw


def _mm_scale(x, w, degp):
    return pl.pallas_call(
        _mm_scale_body,
        grid=(N // R,),
        in_specs=[
            pl.BlockSpec((R, D), lambda i: (i, 0)),
            pl.BlockSpec((D, D), lambda i: (0, 0)),
            pl.BlockSpec((R, NC), lambda i: (i, 0)),
        ],
        out_specs=pl.BlockSpec((R, D), lambda i: (i, 0)),
        out_shape=jax.ShapeDtypeStruct((N, D), jnp.float32),
    )(x, w, degp)


def _mid_body(p_ref, xws_ref, degp_ref, b_ref, w2_ref, o_ref):
    d = degp_ref[:, 0] + degp_ref[:, 1] + 1.0
    dinv = lax.rsqrt(d)
    ssum = p_ref[0] + p_ref[1] + xws_ref[...]
    h = jnp.maximum(dinv[:, None] * ssum + b_ref[0, :][None, :], 0.0)
    hw = jnp.dot(h, w2_ref[...], preferred_element_type=jnp.float32,
                 precision=lax.Precision.HIGHEST)
    o_ref[...] = dinv[:, None] * hw


def _mid(p, xws, degp, b1, w2):
    return pl.pallas_call(
        _mid_body,
        grid=(N // R,),
        in_specs=[
            pl.BlockSpec((NC, R, D), lambda i: (0, i, 0)),
            pl.BlockSpec((R, D), lambda i: (i, 0)),
            pl.BlockSpec((R, NC), lambda i: (i, 0)),
            pl.BlockSpec((1, D), lambda i: (0, 0)),
            pl.BlockSpec((D, D), lambda i: (0, 0)),
        ],
        out_specs=pl.BlockSpec((R, D), lambda i: (i, 0)),
        out_shape=jax.ShapeDtypeStruct((N, D), jnp.float32),
    )(p, xws, degp, b1, w2)


def _final_body(q_ref, xws2_ref, degp_ref, b_ref, o_ref):
    d = degp_ref[:, 0] + degp_ref[:, 1] + 1.0
    dinv = lax.rsqrt(d)
    ssum = q_ref[0] + q_ref[1] + xws2_ref[...]
    o_ref[...] = jnp.maximum(dinv[:, None] * ssum + b_ref[0, :][None, :], 0.0)


def _final(q, xws2, degp, b2):
    return pl.pallas_call(
        _final_body,
        grid=(N // R,),
        in_specs=[
            pl.BlockSpec((NC, R, D), lambda i: (0, i, 0)),
            pl.BlockSpec((R, D), lambda i: (i, 0)),
            pl.BlockSpec((R, NC), lambda i: (i, 0)),
            pl.BlockSpec((1, D), lambda i: (0, 0)),
        ],
        out_specs=pl.BlockSpec((R, D), lambda i: (i, 0)),
        out_shape=jax.ShapeDtypeStruct((N, D), jnp.float32),
    )(q, xws2, degp, b2)


def kernel(x, edge_index, W1, b1, W2, b2):
    pad = CHP * 128 - E
    pad_src = jnp.arange(pad, dtype=jnp.int32) % N
    pad_dst = N + (jnp.arange(pad, dtype=jnp.int32) % 128)
    src2 = jnp.concatenate([edge_index[0], pad_src]).reshape(CHP, 128)
    dst2 = jnp.concatenate([edge_index[1], pad_dst]).reshape(CHP, 128)
    b1r = b1.reshape(1, D)
    b2r = b2.reshape(1, D)

    degp = _degree_partials(dst2).reshape(NC, N).T
    xws1 = _mm_scale(x, W1, degp)
    p = _propagate(xws1, src2, dst2)
    xws2 = _mid(p, xws1, degp, b1r, W2)
    q = _propagate(xws2, src2, dst2)
    return _final(q, xws2, degp, b2r)


# speedup vs baseline: 3.0547x; 1.0665x over previous
import functools

import jax
import jax.numpy as jnp
from jax import lax
from jax.experimental import pallas as pl
from jax.experimental.pallas import tpu as pltpu
from jax.experimental.pallas import tpu_sc as plsc

N = 10000
D = 128
E = 320000
NC, NS = 2, 16
NW = NC * NS
CPW = 80
CPH = CPW // 2
CHP = NW * CPW
N_PAD = N + 128
SL = 640
SL_LAST = N - SL * (NS - 1)

_mesh = lambda: plsc.VectorSubcoreMesh(core_axis_name="c", subcore_axis_name="s")


def _for_my_slice(s, fn):
    @pl.when(s < NS - 1)
    def _():
        def b(k, carry):
            fn(pl.multiple_of(s * SL + k * 128, 128), 128)
            return carry

        lax.fori_loop(0, SL // 128, b, 0)

    @pl.when(s == NS - 1)
    def _():
        base = (NS - 1) * SL
        for k in range(SL_LAST // 128):
            fn(base + k * 128, 128)
        fn(base + (SL_LAST // 128) * 128, SL_LAST % 128)


def _degree_partials(dst2):
    @functools.partial(
        pl.kernel,
        out_type=jax.ShapeDtypeStruct((NC, 1, N), jnp.float32),
        mesh=_mesh(),
        scratch_types=[
            pltpu.VMEM_SHARED((N_PAD,), jnp.float32),
            pltpu.VMEM((CPW, 128), jnp.int32),
            pltpu.VMEM((128,), jnp.float32),
            pltpu.VMEM((128,), jnp.float32),
            pltpu.SemaphoreType.DMA,
        ],
    )
    def k(dst_hbm, out_hbm, acc, didx_all, ones_v, stage, sem):
        c = lax.axis_index("c")
        s = lax.axis_index("s")
        w = s * NC + c
        for i in range(8):
            ones_v[pl.ds(i * 16, 16)] = jnp.ones((16,), jnp.float32)
            stage[pl.ds(i * 16, 16)] = jnp.zeros((16,), jnp.float32)

        _for_my_slice(s, lambda off, ln: pltpu.sync_copy(
            stage.at[pl.ds(0, ln)], acc.at[pl.ds(off, ln)]))
        plsc.subcore_barrier()

        base = w * CPW
        pltpu.sync_copy(dst_hbm.at[pl.ds(base, CPW)], didx_all)

        def body(j, carry):
            pltpu.async_copy(ones_v, acc.at[didx_all.at[j]], sem, add=True)
            return carry

        lax.fori_loop(0, CPW, body, 0)

        def drain(j, carry):
            pltpu.make_async_copy(ones_v, acc.at[didx_all.at[0]], sem).wait()
            return carry

        lax.fori_loop(0, CPW, drain, 0)
        plsc.subcore_barrier()

        def wb(off, ln):
            pltpu.sync_copy(acc.at[pl.ds(off, ln)], stage.at[pl.ds(0, ln)])
            pltpu.sync_copy(stage.at[pl.ds(0, ln)],
                            out_hbm.at[c].at[0].at[pl.ds(off, ln)])

        _for_my_slice(s, wb)

    return k(dst2)


def _propagate(xws, src2, dst2):
    @functools.partial(
        pl.kernel,
        out_type=jax.ShapeDtypeStruct((NC, N, D), jnp.float32),
        mesh=_mesh(),
        scratch_types=[
            pltpu.VMEM_SHARED((N_PAD, D), jnp.float32),
            pltpu.VMEM((128, D), jnp.float32),
            pltpu.VMEM((128, D), jnp.float32),
        ] + [pltpu.VMEM((128,), jnp.int32)] * 8 + [
            pltpu.SemaphoreType.DMA,
            pltpu.SemaphoreType.DMA,
            pltpu.SemaphoreType.DMA,
            pltpu.SemaphoreType.DMA,
            pltpu.SemaphoreType.DMA,
            pltpu.SemaphoreType.DMA,
        ],
    )
    def k(xws_hbm, src_hbm, dst_hbm, out_hbm, acc, rows_a, rows_b,
          sa0, sb0, da0, db0, sa1, sb1, da1, db1,
          gsem_a, gsem_b, ssem_a, ssem_b, isem0, isem1):
        c = lax.axis_index("c")
        s = lax.axis_index("s")
        w = s * NC + c

        def zrow(i, carry):
            def zcol(j, c2):
                rows_a[i, pl.ds(j * 16, 16)] = jnp.zeros((16,), jnp.float32)
                return c2

            return lax.fori_loop(0, D // 16, zcol, carry)

        lax.fori_loop(0, 128, zrow, 0)

        _for_my_slice(s, lambda off, ln: pltpu.sync_copy(
            rows_a.at[pl.ds(0, ln)], acc.at[pl.ds(off, ln)]))
        plsc.subcore_barrier()

        base = w * CPW

        def gath(rows, sidx, sem):
            pltpu.async_copy(xws_hbm.at[sidx], rows, sem)

        def gath_wait(rows, sidx, sem):
            pltpu.make_async_copy(xws_hbm.at[sidx], rows, sem).wait()

        def scat(rows, didx, sem):
            pltpu.async_copy(rows, acc.at[didx], sem, add=True)

        def scat_wait(rows, didx, sem):
            pltpu.make_async_copy(rows, acc.at[didx], sem).wait()

        def idx_fetch(j0, sa, sb, da, db, isem):
            pltpu.async_copy(src_hbm.at[j0], sa, isem)
            pltpu.async_copy(dst_hbm.at[j0], da, isem)
            pltpu.async_copy(src_hbm.at[j0 + 1], sb, isem)
            pltpu.async_copy(dst_hbm.at[j0 + 1], db, isem)

        def idx_wait(sa, sb, da, db, isem):
            pltpu.make_async_copy(src_hbm.at[base], sa, isem).wait()
            pltpu.make_async_copy(dst_hbm.at[base], da, isem).wait()
            pltpu.make_async_copy(src_hbm.at[base], sb, isem).wait()
            pltpu.make_async_copy(dst_hbm.at[base], db, isem).wait()

        NPAIR = CPW // 2

        def pair_body(u, first, last, sa, sb, da, db, isem,
                      nsa, nsb, nda, ndb, nisem):
            j0 = base + 2 * u
            if first:
                @pl.when(u > 0)
                def _():
                    scat_wait(rows_b, db, ssem_b)
            else:
                scat_wait(rows_b, db, ssem_b)
            if last:
                @pl.when(u + 1 < NPAIR)
                def _():
                    idx_fetch(j0 + 2, nsa, nsb, nda, ndb, nisem)
            else:
                idx_fetch(j0 + 2, nsa, nsb, nda, ndb, nisem)
            idx_wait(sa, sb, da, db, isem)
            gath(rows_a, sa, gsem_a)
            gath(rows_b, sb, gsem_b)
            gath_wait(rows_a, sa, gsem_a)
            scat(rows_a, da, ssem_a)
            gath_wait(rows_b, sb, gsem_b)
            scat_wait(rows_a, da, ssem_a)
            scat(rows_b, db, ssem_b)

        idx_fetch(base, sa0, sb0, da0, db0, isem0)

        def outer(t2, carry):
            pair_body(2 * t2, True, False, sa0, sb0, da0, db0, isem0,
                      sa1, sb1, da1, db1, isem1)
            pair_body(2 * t2 + 1, False, True, sa1, sb1, da1, db1, isem1,
                      sa0, sb0, da0, db0, isem0)
            return carry

        lax.fori_loop(0, NPAIR // 2, outer, 0)
        scat_wait(rows_b, db1, ssem_b)
        plsc.subcore_barrier()

        def wb(off, ln):
            pltpu.sync_copy(acc.at[pl.ds(off, ln)], rows_a.at[pl.ds(0, ln)])
            pltpu.sync_copy(rows_a.at[pl.ds(0, ln)],
                            out_hbm.at[c].at[pl.ds(off, ln)])

        _for_my_slice(s, wb)

    return k(xws, src2, dst2)


R = 1000


def _mm_scale_body(x_ref, w_ref, degp_ref, o_ref):
    d = degp_ref[:, 0] + degp_ref[:, 1] + 1.0
    dinv = lax.rsqrt(d)
    xw = jnp.dot(x_ref[...], w_ref[...], preferred_element_type=jnp.float32,
                 precision=lax.Precision.HIGHEST)
    o_ref[...] = dinv[:, None] * xw


def _mm_scale(x, w, degp):
    return pl.pallas_call(
        _mm_scale_body,
        grid=(N // R,),
        in_specs=[
            pl.BlockSpec((R, D), lambda i: (i, 0)),
            pl.BlockSpec((D, D), lambda i: (0, 0)),
            pl.BlockSpec((R, NC), lambda i: (i, 0)),
        ],
        out_specs=pl.BlockSpec((R, D), lambda i: (i, 0)),
        out_shape=jax.ShapeDtypeStruct((N, D), jnp.float32),
    )(x, w, degp)


def _mid_body(p_ref, xws_ref, degp_ref, b_ref, w2_ref, o_ref):
    d = degp_ref[:, 0] + degp_ref[:, 1] + 1.0
    dinv = lax.rsqrt(d)
    ssum = p_ref[0] + p_ref[1] + xws_ref[...]
    h = jnp.maximum(dinv[:, None] * ssum + b_ref[0, :][None, :], 0.0)
    hw = jnp.dot(h, w2_ref[...], preferred_element_type=jnp.float32,
                 precision=lax.Precision.HIGHEST)
    o_ref[...] = dinv[:, None] * hw


def _mid(p, xws, degp, b1, w2):
    return pl.pallas_call(
        _mid_body,
        grid=(N // R,),
        in_specs=[
            pl.BlockSpec((NC, R, D), lambda i: (0, i, 0)),
            pl.BlockSpec((R, D), lambda i: (i, 0)),
            pl.BlockSpec((R, NC), lambda i: (i, 0)),
            pl.BlockSpec((1, D), lambda i: (0, 0)),
            pl.BlockSpec((D, D), lambda i: (0, 0)),
        ],
        out_specs=pl.BlockSpec((R, D), lambda i: (i, 0)),
        out_shape=jax.ShapeDtypeStruct((N, D), jnp.float32),
    )(p, xws, degp, b1, w2)


def _final_body(q_ref, xws2_ref, degp_ref, b_ref, o_ref):
    d = degp_ref[:, 0] + degp_ref[:, 1] + 1.0
    dinv = lax.rsqrt(d)
    ssum = q_ref[0] + q_ref[1] + xws2_ref[...]
    o_ref[...] = jnp.maximum(dinv[:, None] * ssum + b_ref[0, :][None, :], 0.0)


def _final(q, xws2, degp, b2):
    return pl.pallas_call(
        _final_body,
        grid=(N // R,),
        in_specs=[
            pl.BlockSpec((NC, R, D), lambda i: (0, i, 0)),
            pl.BlockSpec((R, D), lambda i: (i, 0)),
            pl.BlockSpec((R, NC), lambda i: (i, 0)),
            pl.BlockSpec((1, D), lambda i: (0, 0)),
        ],
        out_specs=pl.BlockSpec((R, D), lambda i: (i, 0)),
        out_shape=jax.ShapeDtypeStruct((N, D), jnp.float32),
    )(q, xws2, degp, b2)


def kernel(x, edge_index, W1, b1, W2, b2):
    pad = CHP * 128 - E
    pad_src = jnp.arange(pad, dtype=jnp.int32) % N
    pad_dst = N + (jnp.arange(pad, dtype=jnp.int32) % 128)
    src2 = jnp.concatenate([edge_index[0], pad_src]).reshape(CHP, 128)
    dst2 = jnp.concatenate([edge_index[1], pad_dst]).reshape(CHP, 128)
    b1r = b1.reshape(1, D)
    b2r = b2.reshape(1, D)

    degp = _degree_partials(dst2).reshape(NC, N).T
    xws1 = _mm_scale(x, W1, degp)
    p = _propagate(xws1, src2, dst2)
    xws2 = _mid(p, xws1, degp, b1r, W2)
    q = _propagate(xws2, src2, dst2)
    return _final(q, xws2, degp, b2r)


# speedup vs baseline: 3.4285x; 1.1224x over previous
import functools

import jax
import jax.numpy as jnp
from jax import lax
from jax.experimental import pallas as pl
from jax.experimental.pallas import tpu as pltpu
from jax.experimental.pallas import tpu_sc as plsc

N = 10000
D = 128
E = 320000
NC, NS = 2, 16
NW = NC * NS
CPW = 80
CPH = CPW // 2
CHP = NW * CPW
N_PAD = N + 128
SL = 640
SL_LAST = N - SL * (NS - 1)

_mesh = lambda: plsc.VectorSubcoreMesh(core_axis_name="c", subcore_axis_name="s")


def _for_my_slice(s, fn):
    @pl.when(s < NS - 1)
    def _():
        def b(k, carry):
            fn(pl.multiple_of(s * SL + k * 128, 128), 128)
            return carry

        lax.fori_loop(0, SL // 128, b, 0)

    @pl.when(s == NS - 1)
    def _():
        base = (NS - 1) * SL
        for k in range(SL_LAST // 128):
            fn(base + k * 128, 128)
        fn(base + (SL_LAST // 128) * 128, SL_LAST % 128)


def _degree_partials(dst2):
    @functools.partial(
        pl.kernel,
        out_type=jax.ShapeDtypeStruct((NC, 1, N), jnp.float32),
        mesh=_mesh(),
        scratch_types=[
            pltpu.VMEM_SHARED((N_PAD,), jnp.float32),
            pltpu.VMEM((CPW, 128), jnp.int32),
            pltpu.VMEM((128,), jnp.float32),
            pltpu.VMEM((128,), jnp.float32),
            pltpu.SemaphoreType.DMA,
        ],
    )
    def k(dst_hbm, out_hbm, acc, didx_all, ones_v, stage, sem):
        c = lax.axis_index("c")
        s = lax.axis_index("s")
        w = s * NC + c
        for i in range(8):
            ones_v[pl.ds(i * 16, 16)] = jnp.ones((16,), jnp.float32)
            stage[pl.ds(i * 16, 16)] = jnp.zeros((16,), jnp.float32)

        _for_my_slice(s, lambda off, ln: pltpu.sync_copy(
            stage.at[pl.ds(0, ln)], acc.at[pl.ds(off, ln)]))
        plsc.subcore_barrier()

        base = w * CPW
        pltpu.sync_copy(dst_hbm.at[pl.ds(base, CPW)], didx_all)

        def body(j, carry):
            pltpu.async_copy(ones_v, acc.at[didx_all.at[j]], sem, add=True)
            return carry

        lax.fori_loop(0, CPW, body, 0)

        def drain(j, carry):
            pltpu.make_async_copy(ones_v, acc.at[didx_all.at[0]], sem).wait()
            return carry

        lax.fori_loop(0, CPW, drain, 0)
        plsc.subcore_barrier()

        def wb(off, ln):
            pltpu.sync_copy(acc.at[pl.ds(off, ln)], stage.at[pl.ds(0, ln)])
            pltpu.sync_copy(stage.at[pl.ds(0, ln)],
                            out_hbm.at[c].at[0].at[pl.ds(off, ln)])

        _for_my_slice(s, wb)

    return k(dst2)


def _propagate(xws, src2, dst2):
    @functools.partial(
        pl.kernel,
        out_type=jax.ShapeDtypeStruct((NC, N, D), jnp.float32),
        mesh=_mesh(),
        scratch_types=[
            pltpu.VMEM_SHARED((N_PAD, D), jnp.float32),
            pltpu.VMEM((128, D), jnp.float32),
            pltpu.VMEM((128, D), jnp.float32),
        ] + [pltpu.VMEM((128,), jnp.int32)] * 8 + [
            pltpu.SemaphoreType.DMA,
            pltpu.SemaphoreType.DMA,
            pltpu.SemaphoreType.DMA,
            pltpu.SemaphoreType.DMA,
            pltpu.SemaphoreType.DMA,
            pltpu.SemaphoreType.DMA,
        ],
    )
    def k(xws_hbm, src_hbm, dst_hbm, out_hbm, acc, rows_a, rows_b,
          sa0, sb0, da0, db0, sa1, sb1, da1, db1,
          gsem_a, gsem_b, ssem_a, ssem_b, isem0, isem1):
        c = lax.axis_index("c")
        s = lax.axis_index("s")
        w = s * NC + c

        def zrow(i, carry):
            def zcol(j, c2):
                rows_a[i, pl.ds(j * 16, 16)] = jnp.zeros((16,), jnp.float32)
                return c2

            return lax.fori_loop(0, D // 16, zcol, carry)

        lax.fori_loop(0, 128, zrow, 0)

        _for_my_slice(s, lambda off, ln: pltpu.sync_copy(
            rows_a.at[pl.ds(0, ln)], acc.at[pl.ds(off, ln)]))
        plsc.subcore_barrier()

        base = w * CPW

        def gath(rows, sidx, sem):
            pltpu.async_copy(xws_hbm.at[sidx], rows, sem)

        def gath_wait(rows, sidx, sem):
            pltpu.make_async_copy(xws_hbm.at[sidx], rows, sem).wait()

        def scat(rows, didx, sem):
            pltpu.async_copy(rows, acc.at[didx], sem, add=True)

        def scat_wait(rows, didx, sem):
            pltpu.make_async_copy(rows, acc.at[didx], sem).wait()

        def idx_fetch(j0, sa, sb, da, db, isem):
            pltpu.async_copy(src_hbm.at[j0], sa, isem)
            pltpu.async_copy(dst_hbm.at[j0], da, isem)
            pltpu.async_copy(src_hbm.at[j0 + 1], sb, isem)
            pltpu.async_copy(dst_hbm.at[j0 + 1], db, isem)

        def idx_wait(sa, sb, da, db, isem):
            pltpu.make_async_copy(src_hbm.at[base], sa, isem).wait()
            pltpu.make_async_copy(dst_hbm.at[base], da, isem).wait()
            pltpu.make_async_copy(src_hbm.at[base], sb, isem).wait()
            pltpu.make_async_copy(dst_hbm.at[base], db, isem).wait()

        NPAIR = CPW // 2

        def pair_body(u, first, last, sa, sb, da, db, isem,
                      nsa, nsb, nda, ndb, nisem):
            j0 = base + 2 * u
            if first:
                @pl.when(u > 0)
                def _():
                    scat_wait(rows_b, db, ssem_b)
            else:
                scat_wait(rows_b, db, ssem_b)
            if last:
                @pl.when(u + 1 < NPAIR)
                def _():
                    idx_fetch(j0 + 2, nsa, nsb, nda, ndb, nisem)
            else:
                idx_fetch(j0 + 2, nsa, nsb, nda, ndb, nisem)
            gath_wait(rows_a, sa, gsem_a)
            scat(rows_a, da, ssem_a)
            gath(rows_b, sb, gsem_b)
            gath_wait(rows_b, sb, gsem_b)
            scat(rows_b, db, ssem_b)
            scat_wait(rows_a, da, ssem_a)
            if last:
                @pl.when(u + 1 < NPAIR)
                def _():
                    idx_wait(nsa, nsb, nda, ndb, nisem)
                    gath(rows_a, nsa, gsem_a)
            else:
                idx_wait(nsa, nsb, nda, ndb, nisem)
                gath(rows_a, nsa, gsem_a)

        idx_fetch(base, sa0, sb0, da0, db0, isem0)
        idx_wait(sa0, sb0, da0, db0, isem0)
        gath(rows_a, sa0, gsem_a)

        def outer(t2, carry):
            pair_body(2 * t2, True, False, sa0, sb0, da0, db0, isem0,
                      sa1, sb1, da1, db1, isem1)
            pair_body(2 * t2 + 1, False, True, sa1, sb1, da1, db1, isem1,
                      sa0, sb0, da0, db0, isem0)
            return carry

        lax.fori_loop(0, NPAIR // 2, outer, 0)
        scat_wait(rows_b, db1, ssem_b)
        plsc.subcore_barrier()

        def wb(off, ln):
            pltpu.sync_copy(acc.at[pl.ds(off, ln)], rows_a.at[pl.ds(0, ln)])
            pltpu.sync_copy(rows_a.at[pl.ds(0, ln)],
                            out_hbm.at[c].at[pl.ds(off, ln)])

        _for_my_slice(s, wb)

    return k(xws, src2, dst2)


R = 1000


def _mm_scale_body(x_ref, w_ref, degp_ref, o_ref):
    d = degp_ref[:, 0] + degp_ref[:, 1] + 1.0
    dinv = lax.rsqrt(d)
    xw = jnp.dot(x_ref[...], w_ref[...], preferred_element_type=jnp.float32,
                 precision=lax.Precision.HIGHEST)
    o_ref[...] = dinv[:, None] * xw


def _mm_scale(x, w, degp):
    return pl.pallas_call(
        _mm_scale_body,
        grid=(N // R,),
        in_specs=[
            pl.BlockSpec((R, D), lambda i: (i, 0)),
            pl.BlockSpec((D, D), lambda i: (0, 0)),
            pl.BlockSpec((R, NC), lambda i: (i, 0)),
        ],
        out_specs=pl.BlockSpec((R, D), lambda i: (i, 0)),
        out_shape=jax.ShapeDtypeStruct((N, D), jnp.float32),
    )(x, w, degp)


def _mid_body(p_ref, xws_ref, degp_ref, b_ref, w2_ref, o_ref):
    d = degp_ref[:, 0] + degp_ref[:, 1] + 1.0
    dinv = lax.rsqrt(d)
    ssum = p_ref[0] + p_ref[1] + xws_ref[...]
    h = jnp.maximum(dinv[:, None] * ssum + b_ref[0, :][None, :], 0.0)
    hw = jnp.dot(h, w2_ref[...], preferred_element_type=jnp.float32,
                 precision=lax.Precision.HIGHEST)
    o_ref[...] = dinv[:, None] * hw


def _mid(p, xws, degp, b1, w2):
    return pl.pallas_call(
        _mid_body,
        grid=(N // R,),
        in_specs=[
            pl.BlockSpec((NC, R, D), lambda i: (0, i, 0)),
            pl.BlockSpec((R, D), lambda i: (i, 0)),
            pl.BlockSpec((R, NC), lambda i: (i, 0)),
            pl.BlockSpec((1, D), lambda i: (0, 0)),
            pl.BlockSpec((D, D), lambda i: (0, 0)),
        ],
        out_specs=pl.BlockSpec((R, D), lambda i: (i, 0)),
        out_shape=jax.ShapeDtypeStruct((N, D), jnp.float32),
    )(p, xws, degp, b1, w2)


def _final_body(q_ref, xws2_ref, degp_ref, b_ref, o_ref):
    d = degp_ref[:, 0] + degp_ref[:, 1] + 1.0
    dinv = lax.rsqrt(d)
    ssum = q_ref[0] + q_ref[1] + xws2_ref[...]
    o_ref[...] = jnp.maximum(dinv[:, None] * ssum + b_ref[0, :][None, :], 0.0)


def _final(q, xws2, degp, b2):
    return pl.pallas_call(
        _final_body,
        grid=(N // R,),
        in_specs=[
            pl.BlockSpec((NC, R, D), lambda i: (0, i, 0)),
            pl.BlockSpec((R, D), lambda i: (i, 0)),
            pl.BlockSpec((R, NC), lambda i: (i, 0)),
            pl.BlockSpec((1, D), lambda i: (0, 0)),
        ],
        out_specs=pl.BlockSpec((R, D), lambda i: (i, 0)),
        out_shape=jax.ShapeDtypeStruct((N, D), jnp.float32),
    )(q, xws2, degp, b2)


def kernel(x, edge_index, W1, b1, W2, b2):
    pad = CHP * 128 - E
    pad_src = jnp.arange(pad, dtype=jnp.int32) % N
    pad_dst = N + (jnp.arange(pad, dtype=jnp.int32) % 128)
    src2 = jnp.concatenate([edge_index[0], pad_src]).reshape(CHP, 128)
    dst2 = jnp.concatenate([edge_index[1], pad_dst]).reshape(CHP, 128)
    b1r = b1.reshape(1, D)
    b2r = b2.reshape(1, D)

    degp = _degree_partials(dst2).reshape(NC, N).T
    xws1 = _mm_scale(x, W1, degp)
    p = _propagate(xws1, src2, dst2)
    xws2 = _mid(p, xws1, degp, b1r, W2)
    q = _propagate(xws2, src2, dst2)
    return _final(q, xws2, degp, b2r)


# speedup vs baseline: 3.4717x; 1.0126x over previous
import functools

import jax
import jax.numpy as jnp
from jax import lax
from jax.experimental import pallas as pl
from jax.experimental.pallas import tpu as pltpu
from jax.experimental.pallas import tpu_sc as plsc

N = 10000
D = 128
E = 320000
NC, NS = 2, 16
NW = NC * NS
CPW = 80
CPH = CPW // 2
CHP = NW * CPW
N_PAD = N + 128
SL = 640
SL_LAST = N - SL * (NS - 1)

_mesh = lambda: plsc.VectorSubcoreMesh(core_axis_name="c", subcore_axis_name="s")


def _for_my_slice(s, fn):
    @pl.when(s < NS - 1)
    def _():
        def b(k, carry):
            fn(pl.multiple_of(s * SL + k * 128, 128), 128)
            return carry

        lax.fori_loop(0, SL // 128, b, 0)

    @pl.when(s == NS - 1)
    def _():
        base = (NS - 1) * SL
        for k in range(SL_LAST // 128):
            fn(base + k * 128, 128)
        fn(base + (SL_LAST // 128) * 128, SL_LAST % 128)


def _degree_partials(dst2):
    @functools.partial(
        pl.kernel,
        out_type=jax.ShapeDtypeStruct((NC, 1, N), jnp.float32),
        mesh=_mesh(),
        scratch_types=[
            pltpu.VMEM_SHARED((N_PAD,), jnp.float32),
            pltpu.VMEM((CPW, 128), jnp.int32),
            pltpu.VMEM((128,), jnp.float32),
            pltpu.VMEM((128,), jnp.float32),
            pltpu.SemaphoreType.DMA,
        ],
    )
    def k(dst_hbm, out_hbm, acc, didx_all, ones_v, stage, sem):
        c = lax.axis_index("c")
        s = lax.axis_index("s")
        w = s * NC + c
        for i in range(8):
            ones_v[pl.ds(i * 16, 16)] = jnp.ones((16,), jnp.float32)
            stage[pl.ds(i * 16, 16)] = jnp.zeros((16,), jnp.float32)

        _for_my_slice(s, lambda off, ln: pltpu.sync_copy(
            stage.at[pl.ds(0, ln)], acc.at[pl.ds(off, ln)]))
        plsc.subcore_barrier()

        base = w * CPW
        pltpu.sync_copy(dst_hbm.at[pl.ds(base, CPW)], didx_all)

        def body(j, carry):
            pltpu.async_copy(ones_v, acc.at[didx_all.at[j]], sem, add=True)
            return carry

        lax.fori_loop(0, CPW, body, 0)

        def drain(j, carry):
            pltpu.make_async_copy(ones_v, acc.at[didx_all.at[0]], sem).wait()
            return carry

        lax.fori_loop(0, CPW, drain, 0)
        plsc.subcore_barrier()

        def wb(off, ln):
            pltpu.sync_copy(acc.at[pl.ds(off, ln)], stage.at[pl.ds(0, ln)])
            pltpu.sync_copy(stage.at[pl.ds(0, ln)],
                            out_hbm.at[c].at[0].at[pl.ds(off, ln)])

        _for_my_slice(s, wb)

    return k(dst2)


def _propagate(xws, src2, dst2):
    @functools.partial(
        pl.kernel,
        out_type=jax.ShapeDtypeStruct((NC, N, D), jnp.float32),
        mesh=_mesh(),
        scratch_types=[
            pltpu.VMEM_SHARED((N_PAD, D), jnp.float32),
            pltpu.VMEM((128, D), jnp.float32),
            pltpu.VMEM((128, D), jnp.float32),
        ] + [pltpu.VMEM((128,), jnp.int32)] * 8 + [
            pltpu.SemaphoreType.DMA,
            pltpu.SemaphoreType.DMA,
            pltpu.SemaphoreType.DMA,
            pltpu.SemaphoreType.DMA,
            pltpu.SemaphoreType.DMA,
            pltpu.SemaphoreType.DMA,
        ],
    )
    def k(xws_hbm, src_hbm, dst_hbm, out_hbm, acc, rows_a, rows_b,
          sa0, sb0, da0, db0, sa1, sb1, da1, db1,
          gsem_a, gsem_b, ssem_a, ssem_b, isem0, isem1):
        c = lax.axis_index("c")
        s = lax.axis_index("s")
        w = s * NC + c

        def zrow(i, carry):
            def zcol(j, c2):
                rows_a[i, pl.ds(j * 16, 16)] = jnp.zeros((16,), jnp.float32)
                return c2

            return lax.fori_loop(0, D // 16, zcol, carry)

        lax.fori_loop(0, 128, zrow, 0)

        _for_my_slice(s, lambda off, ln: pltpu.async_copy(
            rows_a.at[pl.ds(0, ln)], acc.at[pl.ds(off, ln)], isem0))
        _for_my_slice(s, lambda off, ln: pltpu.make_async_copy(
            rows_a.at[pl.ds(0, ln)], acc.at[pl.ds(off, ln)], isem0).wait())
        plsc.subcore_barrier()

        base = w * CPW

        def gath(rows, sidx, sem):
            pltpu.async_copy(xws_hbm.at[sidx], rows, sem)

        def gath_wait(rows, sidx, sem):
            pltpu.make_async_copy(xws_hbm.at[sidx], rows, sem).wait()

        def scat(rows, didx, sem):
            pltpu.async_copy(rows, acc.at[didx], sem, add=True)

        def scat_wait(rows, didx, sem):
            pltpu.make_async_copy(rows, acc.at[didx], sem).wait()

        def idx_fetch(j0, sa, sb, da, db, isem):
            pltpu.async_copy(src_hbm.at[j0], sa, isem)
            pltpu.async_copy(dst_hbm.at[j0], da, isem)
            pltpu.async_copy(src_hbm.at[j0 + 1], sb, isem)
            pltpu.async_copy(dst_hbm.at[j0 + 1], db, isem)

        def idx_wait(sa, sb, da, db, isem):
            pltpu.make_async_copy(src_hbm.at[base], sa, isem).wait()
            pltpu.make_async_copy(dst_hbm.at[base], da, isem).wait()
            pltpu.make_async_copy(src_hbm.at[base], sb, isem).wait()
            pltpu.make_async_copy(dst_hbm.at[base], db, isem).wait()

        NPAIR = CPW // 2

        def pair_body(u, first, last, sa, sb, da, db, isem,
                      nsa, nsb, nda, ndb, nisem):
            j0 = base + 2 * u
            if first:
                @pl.when(u > 0)
                def _():
                    scat_wait(rows_b, db, ssem_b)
            else:
                scat_wait(rows_b, db, ssem_b)
            if last:
                @pl.when(u + 1 < NPAIR)
                def _():
                    idx_fetch(j0 + 2, nsa, nsb, nda, ndb, nisem)
            else:
                idx_fetch(j0 + 2, nsa, nsb, nda, ndb, nisem)
            gath_wait(rows_a, sa, gsem_a)
            scat(rows_a, da, ssem_a)
            gath(rows_b, sb, gsem_b)
            gath_wait(rows_b, sb, gsem_b)
            scat(rows_b, db, ssem_b)
            scat_wait(rows_a, da, ssem_a)
            if last:
                @pl.when(u + 1 < NPAIR)
                def _():
                    idx_wait(nsa, nsb, nda, ndb, nisem)
                    gath(rows_a, nsa, gsem_a)
            else:
                idx_wait(nsa, nsb, nda, ndb, nisem)
                gath(rows_a, nsa, gsem_a)

        idx_fetch(base, sa0, sb0, da0, db0, isem0)
        idx_wait(sa0, sb0, da0, db0, isem0)
        gath(rows_a, sa0, gsem_a)

        def outer(t2, carry):
            pair_body(2 * t2, True, False, sa0, sb0, da0, db0, isem0,
                      sa1, sb1, da1, db1, isem1)
            pair_body(2 * t2 + 1, False, True, sa1, sb1, da1, db1, isem1,
                      sa0, sb0, da0, db0, isem0)
            return carry

        lax.fori_loop(0, NPAIR // 2, outer, 0)
        scat_wait(rows_b, db1, ssem_b)
        plsc.subcore_barrier()

        def wb_chunks(chunks):
            bufs = (rows_a, rows_b)
            in_sems = (gsem_a, gsem_b)
            out_sems = (ssem_a, ssem_b)
            nch = len(chunks)
            for k, (off, ln) in enumerate(chunks):
                b, isem_k, osem_k = bufs[k % 2], in_sems[k % 2], out_sems[k % 2]
                if k >= 2:
                    off2, ln2 = chunks[k - 2]
                    pltpu.make_async_copy(
                        b.at[pl.ds(0, ln2)],
                        out_hbm.at[c].at[pl.ds(off2, ln2)], osem_k).wait()
                pltpu.async_copy(acc.at[pl.ds(off, ln)], b.at[pl.ds(0, ln)],
                                 isem_k)
                pltpu.make_async_copy(acc.at[pl.ds(off, ln)],
                                      b.at[pl.ds(0, ln)], isem_k).wait()
                pltpu.async_copy(b.at[pl.ds(0, ln)],
                                 out_hbm.at[c].at[pl.ds(off, ln)], osem_k)
            for k in range(max(nch - 2, 0), nch):
                off, ln = chunks[k]
                pltpu.make_async_copy(
                    bufs[k % 2].at[pl.ds(0, ln)],
                    out_hbm.at[c].at[pl.ds(off, ln)], out_sems[k % 2]).wait()

        @pl.when(s < NS - 1)
        def _():
            off0 = pl.multiple_of(s * SL, 128)
            wb_chunks([(off0 + k * 128, 128) for k in range(SL // 128)])

        @pl.when(s == NS - 1)
        def _():
            b15 = (NS - 1) * SL
            wb_chunks([(b15 + k * 128, 128) for k in range(SL_LAST // 128)]
                      + [(b15 + (SL_LAST // 128) * 128, SL_LAST % 128)])

    return k(xws, src2, dst2)


R = 1000


def _mm_scale_body(x_ref, w_ref, degp_ref, o_ref):
    d = degp_ref[:, 0] + degp_ref[:, 1] + 1.0
    dinv = lax.rsqrt(d)
    xw = jnp.dot(x_ref[...], w_ref[...], preferred_element_type=jnp.float32,
                 precision=lax.Precision.HIGHEST)
    o_ref[...] = dinv[:, None] * xw


def _mm_scale(x, w, degp):
    return pl.pallas_call(
        _mm_scale_body,
        grid=(N // R,),
        in_specs=[
            pl.BlockSpec((R, D), lambda i: (i, 0)),
            pl.BlockSpec((D, D), lambda i: (0, 0)),
            pl.BlockSpec((R, NC), lambda i: (i, 0)),
        ],
        out_specs=pl.BlockSpec((R, D), lambda i: (i, 0)),
        out_shape=jax.ShapeDtypeStruct((N, D), jnp.float32),
    )(x, w, degp)


def _mid_body(p_ref, xws_ref, degp_ref, b_ref, w2_ref, o_ref):
    d = degp_ref[:, 0] + degp_ref[:, 1] + 1.0
    dinv = lax.rsqrt(d)
    ssum = p_ref[0] + p_ref[1] + xws_ref[...]
    h = jnp.maximum(dinv[:, None] * ssum + b_ref[0, :][None, :], 0.0)
    hw = jnp.dot(h, w2_ref[...], preferred_element_type=jnp.float32,
                 precision=lax.Precision.HIGHEST)
    o_ref[...] = dinv[:, None] * hw


def _mid(p, xws, degp, b1, w2):
    return pl.pallas_call(
        _mid_body,
        grid=(N // R,),
        in_specs=[
            pl.BlockSpec((NC, R, D), lambda i: (0, i, 0)),
            pl.BlockSpec((R, D), lambda i: (i, 0)),
            pl.BlockSpec((R, NC), lambda i: (i, 0)),
            pl.BlockSpec((1, D), lambda i: (0, 0)),
            pl.BlockSpec((D, D), lambda i: (0, 0)),
        ],
        out_specs=pl.BlockSpec((R, D), lambda i: (i, 0)),
        out_shape=jax.ShapeDtypeStruct((N, D), jnp.float32),
    )(p, xws, degp, b1, w2)


def _final_body(q_ref, xws2_ref, degp_ref, b_ref, o_ref):
    d = degp_ref[:, 0] + degp_ref[:, 1] + 1.0
    dinv = lax.rsqrt(d)
    ssum = q_ref[0] + q_ref[1] + xws2_ref[...]
    o_ref[...] = jnp.maximum(dinv[:, None] * ssum + b_ref[0, :][None, :], 0.0)


def _final(q, xws2, degp, b2):
    return pl.pallas_call(
        _final_body,
        grid=(N // R,),
        in_specs=[
            pl.BlockSpec((NC, R, D), lambda i: (0, i, 0)),
            pl.BlockSpec((R, D), lambda i: (i, 0)),
            pl.BlockSpec((R, NC), lambda i: (i, 0)),
            pl.BlockSpec((1, D), lambda i: (0, 0)),
        ],
        out_specs=pl.BlockSpec((R, D), lambda i: (i, 0)),
        out_shape=jax.ShapeDtypeStruct((N, D), jnp.float32),
    )(q, xws2, degp, b2)


def kernel(x, edge_index, W1, b1, W2, b2):
    pad = CHP * 128 - E
    pad_src = jnp.arange(pad, dtype=jnp.int32) % N
    pad_dst = N + (jnp.arange(pad, dtype=jnp.int32) % 128)
    src2 = jnp.concatenate([edge_index[0], pad_src]).reshape(CHP, 128)
    dst2 = jnp.concatenate([edge_index[1], pad_dst]).reshape(CHP, 128)
    b1r = b1.reshape(1, D)
    b2r = b2.reshape(1, D)

    degp = _degree_partials(dst2).reshape(NC, N).T
    xws1 = _mm_scale(x, W1, degp)
    p = _propagate(xws1, src2, dst2)
    xws2 = _mid(p, xws1, degp, b1r, W2)
    q = _propagate(xws2, src2, dst2)
    return _final(q, xws2, degp, b2r)


# speedup vs baseline: 3.5401x; 1.0197x over previous
import functools

import jax
import jax.numpy as jnp
from jax import lax
from jax.experimental import pallas as pl
from jax.experimental.pallas import tpu as pltpu
from jax.experimental.pallas import tpu_sc as plsc

N = 10000
D = 128
E = 320000
NC, NS = 2, 16
NW = NC * NS
CPW = 80
CPH = CPW // 2
CHP = NW * CPW
N_PAD = N + 128
SL = 640
SL_LAST = N - SL * (NS - 1)

_mesh = lambda: plsc.VectorSubcoreMesh(core_axis_name="c", subcore_axis_name="s")


def _for_my_slice(s, fn):
    @pl.when(s < NS - 1)
    def _():
        def b(k, carry):
            fn(pl.multiple_of(s * SL + k * 128, 128), 128)
            return carry

        lax.fori_loop(0, SL // 128, b, 0)

    @pl.when(s == NS - 1)
    def _():
        base = (NS - 1) * SL
        for k in range(SL_LAST // 128):
            fn(base + k * 128, 128)
        fn(base + (SL_LAST // 128) * 128, SL_LAST % 128)


def _degree_partials(dst2):
    @functools.partial(
        pl.kernel,
        out_type=jax.ShapeDtypeStruct((NC, 1, N), jnp.float32),
        mesh=_mesh(),
        scratch_types=[
            pltpu.VMEM_SHARED((N_PAD,), jnp.float32),
            pltpu.VMEM((CPW, 128), jnp.int32),
            pltpu.VMEM((128,), jnp.float32),
            pltpu.VMEM((128,), jnp.float32),
            pltpu.SemaphoreType.DMA,
        ],
    )
    def k(dst_hbm, out_hbm, acc, didx_all, ones_v, stage, sem):
        c = lax.axis_index("c")
        s = lax.axis_index("s")
        w = s * NC + c
        for i in range(8):
            ones_v[pl.ds(i * 16, 16)] = jnp.ones((16,), jnp.float32)
            stage[pl.ds(i * 16, 16)] = jnp.zeros((16,), jnp.float32)

        _for_my_slice(s, lambda off, ln: pltpu.sync_copy(
            stage.at[pl.ds(0, ln)], acc.at[pl.ds(off, ln)]))
        plsc.subcore_barrier()

        base = w * CPW
        pltpu.sync_copy(dst_hbm.at[pl.ds(base, CPW)], didx_all)

        def body(j, carry):
            pltpu.async_copy(ones_v, acc.at[didx_all.at[j]], sem, add=True)
            return carry

        lax.fori_loop(0, CPW, body, 0)

        def drain(j, carry):
            pltpu.make_async_copy(ones_v, acc.at[didx_all.at[0]], sem).wait()
            return carry

        lax.fori_loop(0, CPW, drain, 0)
        plsc.subcore_barrier()

        def wb(off, ln):
            pltpu.sync_copy(acc.at[pl.ds(off, ln)], stage.at[pl.ds(0, ln)])
            pltpu.sync_copy(stage.at[pl.ds(0, ln)],
                            out_hbm.at[c].at[0].at[pl.ds(off, ln)])

        _for_my_slice(s, wb)

    return k(dst2)


def _propagate(xws, src2, dst2):
    @functools.partial(
        pl.kernel,
        out_type=jax.ShapeDtypeStruct((NC, N, D), jnp.float32),
        mesh=_mesh(),
        scratch_types=[
            pltpu.VMEM_SHARED((N_PAD, D), jnp.float32),
            pltpu.VMEM((128, D), jnp.float32),
            pltpu.VMEM((128, D), jnp.float32),
        ] + [pltpu.VMEM((128,), jnp.int32)] * 8 + [
            pltpu.SemaphoreType.DMA,
            pltpu.SemaphoreType.DMA,
            pltpu.SemaphoreType.DMA,
            pltpu.SemaphoreType.DMA,
            pltpu.SemaphoreType.DMA,
            pltpu.SemaphoreType.DMA,
        ],
    )
    def k(xws_hbm, src_hbm, dst_hbm, out_hbm, acc, rows_a, rows_b,
          sa0, sb0, da0, db0, sa1, sb1, da1, db1,
          gsem_a, gsem_b, ssem_a, ssem_b, isem0, isem1):
        c = lax.axis_index("c")
        s = lax.axis_index("s")
        w = s * NC + c

        def zrow(i, carry):
            def zcol(j, c2):
                rows_a[i, pl.ds(j * 16, 16)] = jnp.zeros((16,), jnp.float32)
                return c2

            return lax.fori_loop(0, D // 16, zcol, carry)

        lax.fori_loop(0, 128, zrow, 0)

        _for_my_slice(s, lambda off, ln: pltpu.async_copy(
            rows_a.at[pl.ds(0, ln)], acc.at[pl.ds(off, ln)], isem0))
        _for_my_slice(s, lambda off, ln: pltpu.make_async_copy(
            rows_a.at[pl.ds(0, ln)], acc.at[pl.ds(off, ln)], isem0).wait())
        plsc.subcore_barrier()

        base = w * CPW

        def gath(rows, sidx, sem):
            pltpu.async_copy(xws_hbm.at[sidx], rows, sem)

        def gath_wait(rows, sidx, sem):
            pltpu.make_async_copy(xws_hbm.at[sidx], rows, sem).wait()

        def scat(rows, didx, sem):
            pltpu.async_copy(rows, acc.at[didx], sem, add=True)

        def scat_wait(rows, didx, sem):
            pltpu.make_async_copy(rows, acc.at[didx], sem).wait()

        def idx_fetch(j0, sa, sb, da, db, isem):
            pltpu.async_copy(src_hbm.at[j0], sa, isem)
            pltpu.async_copy(dst_hbm.at[j0], da, isem)
            pltpu.async_copy(src_hbm.at[j0 + 1], sb, isem)
            pltpu.async_copy(dst_hbm.at[j0 + 1], db, isem)

        def idx_wait(sa, sb, da, db, isem):
            pltpu.make_async_copy(src_hbm.at[base], sa, isem).wait()
            pltpu.make_async_copy(dst_hbm.at[base], da, isem).wait()
            pltpu.make_async_copy(src_hbm.at[base], sb, isem).wait()
            pltpu.make_async_copy(dst_hbm.at[base], db, isem).wait()

        NPAIR = CPW // 2

        def pair_body(u, first, last, sa, sb, da, db, isem,
                      nsa, nsb, nda, ndb, nisem):
            j0 = base + 2 * u
            if first:
                @pl.when(u > 0)
                def _():
                    scat_wait(rows_b, db, ssem_b)
            else:
                scat_wait(rows_b, db, ssem_b)
            if last:
                @pl.when(u + 1 < NPAIR)
                def _():
                    idx_fetch(j0 + 2, nsa, nsb, nda, ndb, nisem)
            else:
                idx_fetch(j0 + 2, nsa, nsb, nda, ndb, nisem)
            gath_wait(rows_a, sa, gsem_a)
            scat(rows_a, da, ssem_a)
            gath(rows_b, sb, gsem_b)
            gath_wait(rows_b, sb, gsem_b)
            scat(rows_b, db, ssem_b)
            scat_wait(rows_a, da, ssem_a)
            if last:
                @pl.when(u + 1 < NPAIR)
                def _():
                    idx_wait(nsa, nsb, nda, ndb, nisem)
                    gath(rows_a, nsa, gsem_a)
            else:
                idx_wait(nsa, nsb, nda, ndb, nisem)
                gath(rows_a, nsa, gsem_a)

        idx_fetch(base, sa0, sb0, da0, db0, isem0)
        idx_wait(sa0, sb0, da0, db0, isem0)
        gath(rows_a, sa0, gsem_a)

        def outer(t2, carry):
            pair_body(2 * t2, True, False, sa0, sb0, da0, db0, isem0,
                      sa1, sb1, da1, db1, isem1)
            pair_body(2 * t2 + 1, False, True, sa1, sb1, da1, db1, isem1,
                      sa0, sb0, da0, db0, isem0)
            return carry

        lax.fori_loop(0, NPAIR // 2, outer, 0)
        scat_wait(rows_b, db1, ssem_b)
        plsc.subcore_barrier()

        def wb_chunks(chunks):
            bufs = (rows_a, rows_b)
            in_sems = (gsem_a, gsem_b)
            out_sems = (ssem_a, ssem_b)
            nch = len(chunks)
            for k, (off, ln) in enumerate(chunks):
                b, isem_k, osem_k = bufs[k % 2], in_sems[k % 2], out_sems[k % 2]
                if k >= 2:
                    off2, ln2 = chunks[k - 2]
                    pltpu.make_async_copy(
                        b.at[pl.ds(0, ln2)],
                        out_hbm.at[c].at[pl.ds(off2, ln2)], osem_k).wait()
                pltpu.async_copy(acc.at[pl.ds(off, ln)], b.at[pl.ds(0, ln)],
                                 isem_k)
                pltpu.make_async_copy(acc.at[pl.ds(off, ln)],
                                      b.at[pl.ds(0, ln)], isem_k).wait()
                pltpu.async_copy(b.at[pl.ds(0, ln)],
                                 out_hbm.at[c].at[pl.ds(off, ln)], osem_k)
            for k in range(max(nch - 2, 0), nch):
                off, ln = chunks[k]
                pltpu.make_async_copy(
                    bufs[k % 2].at[pl.ds(0, ln)],
                    out_hbm.at[c].at[pl.ds(off, ln)], out_sems[k % 2]).wait()

        @pl.when(s < NS - 1)
        def _():
            off0 = pl.multiple_of(s * SL, 128)
            wb_chunks([(off0 + k * 128, 128) for k in range(SL // 128)])

        @pl.when(s == NS - 1)
        def _():
            b15 = (NS - 1) * SL
            wb_chunks([(b15 + k * 128, 128) for k in range(SL_LAST // 128)]
                      + [(b15 + (SL_LAST // 128) * 128, SL_LAST % 128)])

    return k(xws, src2, dst2)


R = 1000


def _mm_scale_body(x_ref, w_ref, degp_ref, o_ref):
    d = degp_ref[:, 0] + degp_ref[:, 1] + 1.0
    dinv = lax.rsqrt(d)
    xw = jnp.dot(x_ref[...], w_ref[...], preferred_element_type=jnp.float32)
    o_ref[...] = dinv[:, None] * xw


def _mm_scale(x, w, degp):
    return pl.pallas_call(
        _mm_scale_body,
        grid=(N // R,),
        in_specs=[
            pl.BlockSpec((R, D), lambda i: (i, 0)),
            pl.BlockSpec((D, D), lambda i: (0, 0)),
            pl.BlockSpec((R, NC), lambda i: (i, 0)),
        ],
        out_specs=pl.BlockSpec((R, D), lambda i: (i, 0)),
        out_shape=jax.ShapeDtypeStruct((N, D), jnp.float32),
    )(x, w, degp)


def _mid_body(p_ref, xws_ref, degp_ref, b_ref, w2_ref, o_ref):
    d = degp_ref[:, 0] + degp_ref[:, 1] + 1.0
    dinv = lax.rsqrt(d)
    ssum = p_ref[0] + p_ref[1] + xws_ref[...]
    h = jnp.maximum(dinv[:, None] * ssum + b_ref[0, :][None, :], 0.0)
    hw = jnp.dot(h, w2_ref[...], preferred_element_type=jnp.float32)
    o_ref[...] = dinv[:, None] * hw


def _mid(p, xws, degp, b1, w2):
    return pl.pallas_call(
        _mid_body,
        grid=(N // R,),
        in_specs=[
            pl.BlockSpec((NC, R, D), lambda i: (0, i, 0)),
            pl.BlockSpec((R, D), lambda i: (i, 0)),
            pl.BlockSpec((R, NC), lambda i: (i, 0)),
            pl.BlockSpec((1, D), lambda i: (0, 0)),
            pl.BlockSpec((D, D), lambda i: (0, 0)),
        ],
        out_specs=pl.BlockSpec((R, D), lambda i: (i, 0)),
        out_shape=jax.ShapeDtypeStruct((N, D), jnp.float32),
    )(p, xws, degp, b1, w2)


def _final_body(q_ref, xws2_ref, degp_ref, b_ref, o_ref):
    d = degp_ref[:, 0] + degp_ref[:, 1] + 1.0
    dinv = lax.rsqrt(d)
    ssum = q_ref[0] + q_ref[1] + xws2_ref[...]
    o_ref[...] = jnp.maximum(dinv[:, None] * ssum + b_ref[0, :][None, :], 0.0)


def _final(q, xws2, degp, b2):
    return pl.pallas_call(
        _final_body,
        grid=(N // R,),
        in_specs=[
            pl.BlockSpec((NC, R, D), lambda i: (0, i, 0)),
            pl.BlockSpec((R, D), lambda i: (i, 0)),
            pl.BlockSpec((R, NC), lambda i: (i, 0)),
            pl.BlockSpec((1, D), lambda i: (0, 0)),
        ],
        out_specs=pl.BlockSpec((R, D), lambda i: (i, 0)),
        out_shape=jax.ShapeDtypeStruct((N, D), jnp.float32),
    )(q, xws2, degp, b2)


def kernel(x, edge_index, W1, b1, W2, b2):
    pad = CHP * 128 - E
    pad_src = jnp.arange(pad, dtype=jnp.int32) % N
    pad_dst = N + (jnp.arange(pad, dtype=jnp.int32) % 128)
    src2 = jnp.concatenate([edge_index[0], pad_src]).reshape(CHP, 128)
    dst2 = jnp.concatenate([edge_index[1], pad_dst]).reshape(CHP, 128)
    b1r = b1.reshape(1, D)
    b2r = b2.reshape(1, D)

    degp = _degree_partials(dst2).reshape(NC, N).T
    xws1 = _mm_scale(x, W1, degp)
    p = _propagate(xws1, src2, dst2)
    xws2 = _mid(p, xws1, degp, b1r, W2)
    q = _propagate(xws2, src2, dst2)
    return _final(q, xws2, degp, b2r)
